# Initial kernel scaffold; baseline (speedup 1.0000x reference)
import jax
import jax.numpy as jnp
from jax.experimental import pallas as pl


def kernel(feats_in, X_in, feats_out):
    raise NotImplementedError("write your pallas kernel here")


# speedup vs baseline: 3.4591x; 3.4591x over previous
import jax
import jax.numpy as jnp
from jax.experimental import pallas as pl
from jax.experimental.pallas import tpu as pltpu

DEC = 96
W = 192
T = 4096
B = 32
NUM = T - 2 * W
CHUNK = 128
NCHUNK = NUM // CHUNK


def _tc_body(x_ref, out_ref, dists_ref):
    c = pl.program_id(0)

    @pl.when(c < NCHUNK)
    def _compute_chunk():
        keyc = x_ref[:, T - 256:]
        acc = jnp.zeros((B, CHUNK), jnp.float32)
        for jh, njl in ((0, 128), (1, 64)):
            tile = x_ref[:, pl.ds((c + jh) * CHUNK, 256)]
            keyh = pltpu.roll(keyc, (256 - 64 - 128 * jh) % 256, axis=1)

            def body(jl, acc, tile=tile, keyh=keyh):
                sl = (256 - jl) % 256
                win = pltpu.roll(tile, sl, axis=1)[:, :CHUNK]
                kj = pltpu.roll(keyh, sl, axis=1)[:, :1]
                d = win - kj
                return acc + d * d

            acc = jax.lax.fori_loop(0, njl, body, acc)
        dists_ref[:, pl.ds(c * CHUNK, CHUNK)] = acc / W

    @pl.when(c == NCHUNK)
    def _argmin_gather():
        dists = dists_ref[:, :]
        m = jnp.min(dists, axis=1, keepdims=True)
        iota = jax.lax.broadcasted_iota(jnp.int32, (B, NUM), 1)
        idx = jnp.min(jnp.where(dists == m, iota, NUM), axis=1, keepdims=True)
        sh = idx + W
        y = x_ref[:, :]
        for k in range(12):
            amt = 1 << k
            bit = ((sh >> k) & 1) != 0
            rolled = jnp.concatenate([y[:, amt:], y[:, :amt]], axis=1)
            y = jnp.where(bit, rolled, y)
        out_ref[:, :] = y[:, :DEC]


def kernel(feats_in, X_in, feats_out):
    x = X_in[:, :, 0]
    out = pl.pallas_call(
        _tc_body,
        grid=(NCHUNK + 1,),
        in_specs=[pl.BlockSpec((B, T), lambda c: (0, 0))],
        out_specs=pl.BlockSpec((B, DEC), lambda c: (0, 0)),
        out_shape=jax.ShapeDtypeStruct((B, DEC), jnp.float32),
        scratch_shapes=[pltpu.VMEM((B, NUM), jnp.float32)],
    )(x)
    return out[:, :, None]


# speedup vs baseline: 49.8126x; 14.4005x over previous
import jax
import jax.numpy as jnp
from jax.experimental import pallas as pl
from jax.experimental.pallas import tpu as pltpu

DEC = 96
W = 192
T = 4096
B = 32
NUM = T - 2 * W
PADNUM = 3840
G = 3
NGROUP = PADNUM // (G * 128)
TILEW = (G + 1) * 128


def _tc_body(x_ref, out_ref, dists_ref, kb_ref):
    for j in range(W):
        col = x_ref[:, T - W + j : T - W + j + 1]
        kb_ref[:, j * 128 : (j + 1) * 128] = jnp.broadcast_to(col, (B, 128))

    for g in range(NGROUP):
        base = g * G * 128
        accs = [jnp.zeros((B, 128), jnp.float32) for _ in range(G)]
        for jh, njl in ((0, 128), (1, 64)):
            tile = x_ref[:, pl.ds(base + 128 * jh, TILEW)]

            def body(jl, accs, tile=tile, jh=jh):
                sl = (TILEW - jl) % TILEW
                rolled = pltpu.roll(tile, sl, axis=1)
                kjb = kb_ref[:, pl.ds((128 * jh + jl) * 128, 128)]
                out = []
                for s in range(G):
                    d = rolled[:, s * 128 : (s + 1) * 128] - kjb
                    out.append(accs[s] + d * d)
                return out

            accs = jax.lax.fori_loop(0, njl, body, accs, unroll=8)
        for s in range(G):
            dists_ref[:, base + s * 128 : base + (s + 1) * 128] = accs[s] / W

    dists = dists_ref[:, :NUM]
    m = jnp.min(dists, axis=1, keepdims=True)
    iota = jax.lax.broadcasted_iota(jnp.int32, (B, NUM), 1)
    idx = jnp.min(jnp.where(dists == m, iota, NUM), axis=1, keepdims=True)
    sh = idx + W
    y = x_ref[:, :]
    for k in range(12):
        amt = 1 << k
        bit = ((sh >> k) & 1) != 0
        rolled = jnp.concatenate([y[:, amt:], y[:, :amt]], axis=1)
        y = jnp.where(bit, rolled, y)
    out_ref[:, :] = y[:, :DEC]


def kernel(feats_in, X_in, feats_out):
    x = X_in[:, :, 0]
    out = pl.pallas_call(
        _tc_body,
        in_specs=[pl.BlockSpec((B, T), lambda: (0, 0))],
        out_specs=pl.BlockSpec((B, DEC), lambda: (0, 0)),
        out_shape=jax.ShapeDtypeStruct((B, DEC), jnp.float32),
        scratch_shapes=[
            pltpu.VMEM((B, PADNUM), jnp.float32),
            pltpu.VMEM((B, W * 128), jnp.float32),
        ],
    )(x)
    return out[:, :, None]
